# baseline (speedup 1.0000x reference)
import math

import jax
import jax.numpy as jnp
from jax.experimental import pallas as pl
from jax.experimental.pallas import tpu as pltpu

_B, _S, _DIN, _DOUT = 32, 64, 2048, 2048
_BASE_LR = 0.01
_SCALAR_SCALER = math.sqrt(_DIN)
_LR_SCALE = _BASE_LR * math.sqrt(max(_DIN, _DOUT)) * math.sqrt(1.0 / _DIN)

_BO = 1024


def _body(x_ref, log_lr_ref, state_ref, base_ref, o_ref):
    b = pl.program_id(0)
    ob = pl.program_id(1)

    @pl.when(jnp.logical_and(b == 0, ob == 0))
    def _():
        log_lr_ref[...] = _LR_SCALE * jnp.exp(log_lr_ref[...] * _SCALAR_SCALER)

    rows = pl.ds(ob * _BO, _BO)
    w = log_lr_ref[rows, :] * state_ref[0] + base_ref[rows, :]
    o_ref[0] = jax.lax.dot_general(
        x_ref[0], w, (((1,), (1,)), ((), ())),
        preferred_element_type=jnp.float32)


def _call(x, log_lr, state, base_w, interpret=False):
    n_ob = _DOUT // _BO
    return pl.pallas_call(
        _body,
        out_shape=jax.ShapeDtypeStruct((_B, _S, _DOUT), jnp.float32),
        grid=(_B, n_ob),
        in_specs=[
            pl.BlockSpec((1, _S, _DIN), lambda b, ob: (b, 0, 0)),
            pl.BlockSpec((_DOUT, _DIN), lambda b, ob: (0, 0),
                         pipeline_mode=pl.Buffered(1)),
            pl.BlockSpec((1, _BO, _DIN), lambda b, ob: (b, ob, 0)),
            pl.BlockSpec((_DOUT, _DIN), lambda b, ob: (0, 0),
                         pipeline_mode=pl.Buffered(1)),
        ],
        out_specs=pl.BlockSpec((1, _S, _BO), lambda b, ob: (b, 0, ob)),
        compiler_params=pltpu.CompilerParams(
            dimension_semantics=("parallel", "arbitrary"),
            vmem_limit_bytes=56 * 1024 * 1024,
        ),
        name="ittt_linear",
        interpret=interpret,
    )(x, log_lr, state, base_w)


def kernel(x, log_lr, state, momentum, base_w):
    del momentum
    return _call(x, log_lr, state, base_w)
